# baseline (speedup 1.0000x reference)
import functools

import jax
import jax.numpy as jnp
from jax.experimental import pallas as pl

_B, _N, _F_IN, _H, _OUT = 4, 512, 128, 64, 64

_R_WN = 0
_R_WF = 128
_R_W1 = (192, 384)
_R_W2 = (256, 448)
_R_VEC = 576
_R_TOT = 592


def _dot(a, b):
    return jax.lax.dot_general(
        a, b, (((1,), (0,)), ((), ())), preferred_element_type=jnp.float32
    )


def _fwd_kernel(x_ref, p_ref, out_ref):
    ntot = _B * _N
    v0 = _R_VEC
    bn = p_ref[v0 : v0 + 1, : _H]
    x = _dot(x_ref[:], p_ref[_R_WN : _R_WN + _F_IN, : _H]) + bn

    row = jax.lax.broadcasted_iota(jnp.int32, (ntot, 1), 0)
    in_graph = row < _N

    for li in range(2):
        W1 = p_ref[_R_W1[li] : _R_W1[li] + _H, :]
        W2 = p_ref[_R_W2[li] : _R_W2[li] + 2 * _H, : _H]
        vr = v0 + 2 + 6 * li
        h = jnp.maximum(x, 0.0)
        msg = h[: _N, :] + 1e-7
        gate = msg * p_ref[vr : vr + 1, : _H]
        m = jnp.max(gate, axis=0, keepdims=True)
        e = jnp.exp(gate - m)
        denom = jnp.sum(e, axis=0, keepdims=True)
        aggr = jnp.sum(msg * e, axis=0, keepdims=True) / (denom + 1e-16)
        out = h + jnp.where(in_graph, aggr, 0.0)
        hh = _dot(out, W1) + p_ref[vr + 1 : vr + 2, :]
        mu = jnp.mean(hh, axis=-1, keepdims=True)
        var = jnp.mean(hh * hh, axis=-1, keepdims=True) - mu * mu
        hh = (hh - mu) * jax.lax.rsqrt(var + 1e-5)
        hh = hh * p_ref[vr + 2 : vr + 3, :] + p_ref[vr + 3 : vr + 4, :]
        hh = jnp.maximum(hh, 0.0)
        x = x + _dot(hh, W2) + p_ref[vr + 4 : vr + 5, : _H]
    y = jnp.maximum(x, 0.0)
    out_ref[:] = _dot(y, p_ref[_R_WF : _R_WF + _H, : _OUT]) + p_ref[
        v0 + 1 : v0 + 2, : _OUT
    ]


def _pack(Wn, bn, Wf, bf, t0, W1_0, b1_0, g0, be0, W2_0, b2_0,
          t1, W1_1, b1_1, g1, be1, W2_1, b2_1):
    pad_l = lambda a: jnp.pad(a, ((0, 0), (0, 128 - a.shape[1])))
    padv = lambda v: jnp.pad(v, (0, 128 - v.shape[0]))
    zrow = jnp.zeros((128,), jnp.float32)
    vec_rows = [padv(bn), padv(bf)]
    for (t, b1, g, be, b2) in ((t0, b1_0, g0, be0, b2_0),
                               (t1, b1_1, g1, be1, b2_1)):
        vec_rows += [jnp.full((128,), t), b1, g, be, padv(b2), zrow]
    vec_rows += [zrow, zrow]
    return jnp.concatenate(
        [pad_l(Wn), pad_l(Wf), W1_0, pad_l(W2_0), W1_1, pad_l(W2_1),
         jnp.stack(vec_rows)],
        axis=0,
    )


@functools.partial(jax.jit, static_argnames=())
def kernel(batch, Wn, bn, Wf, bf, t0, W1_0, b1_0, g0, be0, W2_0, b2_0,
           t1, W1_1, b1_1, g1, be1, W2_1, b2_1):
    b, n, f = batch.shape
    x = batch.reshape(b * n, f)
    packed = _pack(Wn, bn, Wf, bf, t0, W1_0, b1_0, g0, be0, W2_0, b2_0,
                   t1, W1_1, b1_1, g1, be1, W2_1, b2_1)
    out = pl.pallas_call(
        _fwd_kernel,
        out_shape=jax.ShapeDtypeStruct((b * n, _OUT), jnp.float32),
    )(x, packed)
    return out.reshape(b, n, _OUT)


# speedup vs baseline: 1.6147x; 1.6147x over previous
import functools

import jax
import jax.numpy as jnp
from jax.experimental import pallas as pl

_B, _N, _F_IN, _H, _OUT = 4, 512, 128, 64, 64


def _dot(a, b):
    return jax.lax.dot_general(
        a, b, (((1,), (0,)), ((), ())), preferred_element_type=jnp.float32
    )


def _fwd_kernel(
    x_ref, Wn_ref, bn_ref, Wf_ref, bf_ref,
    t0_ref, W10_ref, b10_ref, g0_ref, be0_ref, W20_ref, b20_ref,
    t1_ref, W11_ref, b11_ref, g1_ref, be1_ref, W21_ref, b21_ref,
    out_ref,
):
    ntot = _B * _N
    x = _dot(x_ref[:], Wn_ref[:]) + bn_ref[:]

    row = jax.lax.broadcasted_iota(jnp.int32, (ntot, 1), 0)
    in_graph = row < _N

    layers = (
        (t0_ref, W10_ref, b10_ref, g0_ref, be0_ref, W20_ref, b20_ref),
        (t1_ref, W11_ref, b11_ref, g1_ref, be1_ref, W21_ref, b21_ref),
    )
    for (t_ref, W1_ref, b1_ref, g_ref, be_ref, W2_ref, b2_ref) in layers:
        h = jnp.maximum(x, 0.0)
        msg = h[: _N, :] + 1e-7
        gate = msg * t_ref[0, 0]
        m = jnp.max(gate, axis=0, keepdims=True)
        e = jnp.exp(gate - m)
        denom = jnp.sum(e, axis=0, keepdims=True)
        aggr = jnp.sum(msg * e, axis=0, keepdims=True) / (denom + 1e-16)
        out = h + jnp.where(in_graph, aggr, 0.0)
        hh = _dot(out, W1_ref[:]) + b1_ref[:]
        mu = jnp.mean(hh, axis=-1, keepdims=True)
        var = jnp.mean(hh * hh, axis=-1, keepdims=True) - mu * mu
        hh = (hh - mu) * jax.lax.rsqrt(var + 1e-5) * g_ref[:] + be_ref[:]
        hh = jnp.maximum(hh, 0.0)
        x = x + _dot(hh, W2_ref[:]) + b2_ref[:]
    y = jnp.maximum(x, 0.0)
    out_ref[:] = _dot(y, Wf_ref[:]) + bf_ref[:]


@functools.partial(jax.jit, static_argnames=())
def kernel(batch, Wn, bn, Wf, bf, t0, W1_0, b1_0, g0, be0, W2_0, b2_0,
           t1, W1_1, b1_1, g1, be1, W2_1, b2_1):
    b, n, f = batch.shape
    x = batch.reshape(b * n, f)
    r2 = lambda v: v.reshape(1, -1)
    out = pl.pallas_call(
        _fwd_kernel,
        out_shape=jax.ShapeDtypeStruct((b * n, _OUT), jnp.float32),
    )(
        x, Wn, r2(bn), Wf, r2(bf),
        t0.reshape(1, 1), W1_0, r2(b1_0), r2(g0), r2(be0), W2_0, r2(b2_0),
        t1.reshape(1, 1), W1_1, r2(b1_1), r2(g1), r2(be1), W2_1, r2(b2_1),
    )
    return out.reshape(b, n, _OUT)
